# baseline (speedup 1.0000x reference)
import jax
import jax.numpy as jnp
from jax.experimental import pallas as pl
from jax.experimental.pallas import tpu as pltpu

_V = 100000
_K = 64
_LAMB = 0.01
_B = 256
_TV = 4000
_NT = _V // _TV


def _fused_kernel(s_ref, p_ref, o_ref, e_ref, acc_ref, score_ref, reg_ref,
                  q_ref):
    i = pl.program_id(0)

    @pl.when(i == 0)
    def _init():
        s = s_ref[...]
        p = p_ref[...]
        o = o_ref[...]
        s0, s1, s2 = s[:, 0, :], s[:, 1, :], s[:, 2, :]
        p0, p1, p2 = p[:, 0, :], p[:, 1, :], p[:, 2, :]
        o0, o1, o2 = o[:, 0, :], o[:, 1, :], o[:, 2, :]
        qo = jnp.concatenate([s2 * p2, s1 * p1, s0 * p0], axis=1)
        qs = jnp.concatenate([p0 * o2, p1 * o1, p2 * o0], axis=1)
        q_ref[0:_B, :] = qo.astype(jnp.bfloat16)
        q_ref[_B:2 * _B, :] = qs.astype(jnp.bfloat16)
        score_ref[...] = jnp.sum(s0 * p0 * o2 + s1 * p1 * o1 + s2 * p2 * o0,
                                 axis=1, keepdims=True)
        reg_ref[...] = (_LAMB / 3.0) * jnp.sum(
            jnp.abs(s) ** 3 + jnp.abs(p) ** 3 + jnp.abs(o) ** 3,
            axis=(1, 2)).reshape(_B, 1)
        acc_ref[...] = jnp.zeros_like(acc_ref)

    logits = jax.lax.dot_general(
        q_ref[...], e_ref[...],
        (((1,), (1,)), ((), ())), preferred_element_type=jnp.float32)
    acc_ref[...] += jnp.sum(jnp.exp(logits), axis=1, keepdims=True)


def kernel(triples, emb):
    sub = triples[:, 0]
    pred = triples[:, 1]
    obj = triples[:, 2]

    s = jnp.take(emb, sub, axis=0)
    p = jnp.take(emb, pred, axis=0)
    o = jnp.take(emb, obj, axis=0)

    e = emb.reshape(_V, 3 * _K).astype(jnp.bfloat16)

    acc, score2, reg2 = pl.pallas_call(
        _fused_kernel,
        grid=(_NT,),
        in_specs=[
            pl.BlockSpec((_B, 3, _K), lambda i: (0, 0, 0)),
            pl.BlockSpec((_B, 3, _K), lambda i: (0, 0, 0)),
            pl.BlockSpec((_B, 3, _K), lambda i: (0, 0, 0)),
            pl.BlockSpec((_TV, 3 * _K), lambda i: (i, 0)),
        ],
        out_specs=[
            pl.BlockSpec((2 * _B, 1), lambda i: (0, 0)),
            pl.BlockSpec((_B, 1), lambda i: (0, 0)),
            pl.BlockSpec((_B, 1), lambda i: (0, 0)),
        ],
        out_shape=[
            jax.ShapeDtypeStruct((2 * _B, 1), jnp.float32),
            jax.ShapeDtypeStruct((_B, 1), jnp.float32),
            jax.ShapeDtypeStruct((_B, 1), jnp.float32),
        ],
        scratch_shapes=[pltpu.VMEM((2 * _B, 3 * _K), jnp.bfloat16)],
    )(s, p, o, e)

    score = score2[:, 0]
    reg = reg2[:, 0]
    es = jnp.exp(score)
    lse_o = jnp.log(acc[:_B, 0] - es)
    lse_s = jnp.log(acc[_B:, 0] - es)
    total_loss = jnp.sum(-2.0 * score + lse_o + lse_s + reg)
    return score, total_loss
